# baseline (speedup 1.0000x reference)
import functools

import jax
import jax.numpy as jnp
from jax import lax
from jax.experimental import pallas as pl
from jax.experimental.pallas import tpu as pltpu
from jax.experimental.pallas import tpu_sc as plsc

_NUM_PIECES = 4096 * 200 * 4
_PF = 16
_NW = 32
_PER_W = _NUM_PIECES // _NW
_GPC = 32
_C = _GPC * 16
_NCHUNK = _PER_W // _C
_NSLOT = 2
_NJ = _NCHUNK // _NSLOT
_G = _PER_W // 16


def _make_kernel():
    mesh = plsc.VectorSubcoreMesh(core_axis_name="c", subcore_axis_name="s")

    @functools.partial(
        pl.kernel,
        mesh=mesh,
        compiler_params=pltpu.CompilerParams(use_tc_tiling_on_sc=False),
        out_type=jax.ShapeDtypeStruct((_NUM_PIECES, _PF), jnp.float32),
        scratch_types=[
            pltpu.VMEM((_G, 16), jnp.int32),
            pltpu.VMEM((_C, _PF), jnp.float32),
            pltpu.VMEM((_C, _PF), jnp.float32),
            pltpu.SemaphoreType.DMA,
            pltpu.SemaphoreType.DMA,
            pltpu.SemaphoreType.DMA,
            pltpu.SemaphoreType.DMA,
        ],
    )
    def emb_kernel(idx_hbm, table_hbm, out_hbm, idx_v, bufa, bufb, g0, g1, w0, w1):
        bufs = (bufa, bufb)
        gsems = (g0, g1)
        wsems = (w0, w1)
        wid = lax.axis_index("s") * 2 + lax.axis_index("c")
        base = wid * _PER_W

        pltpu.sync_copy(idx_hbm.at[wid], idx_v)

        def fire(c, buf, gsem):
            for g in range(_GPC):
                vals = idx_v[c * _GPC + g]
                pltpu.async_copy(
                    table_hbm.at[vals],
                    buf.at[pl.ds(g * 16, 16)],
                    gsem,
                )

        def drain_gather(c, buf, gsem):
            for g in range(_GPC):
                vals = idx_v[c * _GPC]
                pltpu.make_async_copy(
                    table_hbm.at[vals],
                    buf.at[pl.ds(g * 16, 16)],
                    gsem,
                ).wait()

        def body(j, carry):
            for s in range(_NSLOT):
                c = _NSLOT * j + s

                @pl.when(j > 0)
                def _(s=s, wsem=wsems[s], buf=bufs[s]):
                    pltpu.make_async_copy(
                        buf, out_hbm.at[pl.ds(base, _C)], wsem
                    ).wait()

                fire(c, bufs[s], gsems[s])
            for s in range(_NSLOT):
                c = _NSLOT * j + s
                drain_gather(c, bufs[s], gsems[s])
                pltpu.async_copy(
                    bufs[s], out_hbm.at[pl.ds(base + c * _C, _C)], wsems[s]
                )
            return carry

        lax.fori_loop(0, _NJ, body, 0)
        for s in range(_NSLOT):
            pltpu.make_async_copy(
                bufs[s], out_hbm.at[pl.ds(base, _C)], wsems[s]
            ).wait()

    return emb_kernel


_emb = _make_kernel()


def kernel(tokens, embedding):
    idx4 = (tokens.reshape(-1, 1) * 4 + jnp.arange(4, dtype=tokens.dtype)).reshape(
        _NW, _G, 16
    )
    out = _emb(idx4, embedding.reshape(4 * 1000000, _PF))
    return out.reshape(tokens.shape[0], tokens.shape[1], 64)


# speedup vs baseline: 1.2136x; 1.2136x over previous
import functools

import jax
import jax.numpy as jnp
from jax import lax
from jax.experimental import pallas as pl
from jax.experimental.pallas import tpu as pltpu
from jax.experimental.pallas import tpu_sc as plsc

_NUM_TOKENS = 4096 * 200
_FEATURES = 64
_NW = 32
_PER_W = _NUM_TOKENS // _NW
_K = 128
_KSUB = 5
_C = _K * _KSUB
_NCHUNK = _PER_W // _C
_NROWS = _PER_W // _K


def _make_kernel():
    mesh = plsc.VectorSubcoreMesh(core_axis_name="c", subcore_axis_name="s")

    @functools.partial(
        pl.kernel,
        mesh=mesh,
        compiler_params=pltpu.CompilerParams(use_tc_tiling_on_sc=False),
        out_type=jax.ShapeDtypeStruct((_NUM_TOKENS, _FEATURES), jnp.float32),
        scratch_types=[
            pltpu.VMEM((_NROWS, _K), jnp.int32),
            pltpu.VMEM((_C, _FEATURES), jnp.float32),
            pltpu.VMEM((_C, _FEATURES), jnp.float32),
            pltpu.SemaphoreType.DMA,
            pltpu.SemaphoreType.DMA,
        ],
    )
    def emb_kernel(idx_hbm, table_hbm, out_hbm, idx_v, buf0, buf1, sem0, sem1):
        wid = lax.axis_index("s") * 2 + lax.axis_index("c")
        base = wid * _PER_W

        pltpu.sync_copy(idx_hbm.at[wid], idx_v)

        def fire(c, buf, sem):
            for s in range(_KSUB):
                pltpu.async_copy(
                    table_hbm.at[idx_v.at[c * _KSUB + s]],
                    buf.at[pl.ds(s * _K, _K)],
                    sem,
                )

        def drain(buf, sem):
            for s in range(_KSUB):
                pltpu.make_async_copy(
                    table_hbm.at[idx_v.at[0]],
                    buf.at[pl.ds(s * _K, _K)],
                    sem,
                ).wait()

        def write(c, buf):
            pltpu.sync_copy(buf, out_hbm.at[pl.ds(base + c * _C, _C)])

        fire(0, buf0, sem0)
        nj = _NCHUNK // 2

        def body(j, carry):
            fire(2 * j + 1, buf1, sem1)
            drain(buf0, sem0)
            write(2 * j, buf0)

            @pl.when(j < nj - 1)
            def _():
                fire(2 * j + 2, buf0, sem0)

            drain(buf1, sem1)
            write(2 * j + 1, buf1)
            return carry

        lax.fori_loop(0, nj, body, 0)

    return emb_kernel


_emb = _make_kernel()


def kernel(tokens, embedding):
    idx = tokens.reshape(_NW, _NROWS, _K)
    out = _emb(idx, embedding)
    return out.reshape(tokens.shape[0], tokens.shape[1], _FEATURES)


# speedup vs baseline: 1.2137x; 1.0001x over previous
import functools

import jax
import jax.numpy as jnp
from jax import lax
from jax.experimental import pallas as pl
from jax.experimental.pallas import tpu as pltpu
from jax.experimental.pallas import tpu_sc as plsc

_NUM_TOKENS = 4096 * 200
_FEATURES = 64
_NW = 32
_PER_W = _NUM_TOKENS // _NW
_K = 128
_KSUB = 4
_C = _K * _KSUB
_NCHUNK = _PER_W // _C
_NROWS = _PER_W // _K


def _make_kernel():
    mesh = plsc.VectorSubcoreMesh(core_axis_name="c", subcore_axis_name="s")

    @functools.partial(
        pl.kernel,
        mesh=mesh,
        compiler_params=pltpu.CompilerParams(use_tc_tiling_on_sc=False),
        out_type=jax.ShapeDtypeStruct((_NUM_TOKENS, _FEATURES), jnp.float32),
        scratch_types=[
            pltpu.VMEM((_NROWS, _K), jnp.int32),
            pltpu.VMEM((_C, _FEATURES), jnp.float32),
            pltpu.VMEM((_C, _FEATURES), jnp.float32),
            pltpu.SemaphoreType.DMA,
            pltpu.SemaphoreType.DMA,
        ],
    )
    def emb_kernel(idx_hbm, table_hbm, out_hbm, idx_v, buf0, buf1, sem0, sem1):
        wid = lax.axis_index("s") * 2 + lax.axis_index("c")
        base = wid * _PER_W

        pltpu.sync_copy(idx_hbm.at[wid], idx_v)

        def fire(c, buf, sem):
            for s in range(_KSUB):
                pltpu.async_copy(
                    table_hbm.at[idx_v.at[c * _KSUB + s]],
                    buf.at[pl.ds(s * _K, _K)],
                    sem,
                )

        def drain(buf, sem):
            for s in range(_KSUB):
                pltpu.make_async_copy(
                    table_hbm.at[idx_v.at[0]],
                    buf.at[pl.ds(s * _K, _K)],
                    sem,
                ).wait()

        def write(c, buf):
            pltpu.sync_copy(buf, out_hbm.at[pl.ds(base + c * _C, _C)])

        fire(0, buf0, sem0)
        nj = _NCHUNK // 2

        def body(j, carry):
            fire(2 * j + 1, buf1, sem1)
            drain(buf0, sem0)
            write(2 * j, buf0)

            @pl.when(j < nj - 1)
            def _():
                fire(2 * j + 2, buf0, sem0)

            drain(buf1, sem1)
            write(2 * j + 1, buf1)
            return carry

        lax.fori_loop(0, nj, body, 0)

    return emb_kernel


_emb = _make_kernel()


def kernel(tokens, embedding):
    idx = tokens.reshape(_NW, _NROWS, _K)
    out = _emb(idx, embedding)
    return out.reshape(tokens.shape[0], tokens.shape[1], _FEATURES)


# speedup vs baseline: 1.2140x; 1.0003x over previous
import functools

import jax
import jax.numpy as jnp
from jax import lax
from jax.experimental import pallas as pl
from jax.experimental.pallas import tpu as pltpu
from jax.experimental.pallas import tpu_sc as plsc

_NUM_TOKENS = 4096 * 200
_FEATURES = 64
_NW = 32
_PER_W = _NUM_TOKENS // _NW
_K = 128
_KSUB = 4
_C = _K * _KSUB
_NCHUNK = _PER_W // _C
_NROWS = _PER_W // _K


def _make_kernel():
    mesh = plsc.VectorSubcoreMesh(core_axis_name="c", subcore_axis_name="s")

    @functools.partial(
        pl.kernel,
        mesh=mesh,
        compiler_params=pltpu.CompilerParams(use_tc_tiling_on_sc=False),
        out_type=jax.ShapeDtypeStruct((_NUM_TOKENS, _FEATURES), jnp.float32),
        scratch_types=[
            pltpu.VMEM((_NROWS, _K), jnp.int32),
            pltpu.VMEM((_C, _FEATURES), jnp.float32),
            pltpu.VMEM((_C, _FEATURES), jnp.float32),
            pltpu.SemaphoreType.DMA,
            pltpu.SemaphoreType.DMA,
        ],
    )
    def emb_kernel(idx_hbm, table_hbm, out_hbm, idx_v, buf0, buf1, sem0, sem1):
        wid = lax.axis_index("s") * 2 + lax.axis_index("c")
        base = wid * _PER_W

        pltpu.sync_copy(idx_hbm.at[wid], idx_v)

        def fire(c, buf, sem):
            for s in range(_KSUB):
                pltpu.async_copy(
                    table_hbm.at[idx_v.at[c * _KSUB + s]],
                    buf.at[pl.ds(s * _K, _K)],
                    sem,
                )

        def drain(buf, sem):
            for s in range(_KSUB):
                pltpu.make_async_copy(
                    table_hbm.at[idx_v.at[0]],
                    buf.at[pl.ds(s * _K, _K)],
                    sem,
                ).wait()

        def write(c, buf):
            pltpu.sync_copy(buf, out_hbm.at[pl.ds(base + c * _C, _C)])

        fire(0, buf0, sem0)
        nj = _NCHUNK // 2

        def body(j, carry):
            fire(2 * j + 1, buf1, sem1)
            drain(buf0, sem0)
            write(2 * j, buf0)

            @pl.when(j < nj - 1)
            def _():
                fire(2 * j + 2, buf0, sem0)

            drain(buf1, sem1)
            write(2 * j + 1, buf1)
            return carry

        lax.fori_loop(0, nj, body, 0)

    return emb_kernel


_emb_cache = []


def kernel(tokens, embedding):
    if not _emb_cache:
        _emb_cache.append(_make_kernel())
    idx = tokens.reshape(_NW, _NROWS, _K)
    out = _emb_cache[0](idx, embedding)
    return out.reshape(tokens.shape[0], tokens.shape[1], _FEATURES)
